# Initial kernel scaffold; baseline (speedup 1.0000x reference)
import jax
import jax.numpy as jnp
from jax.experimental import pallas as pl


def kernel(x, senders, receivers, batch, p):
    raise NotImplementedError("write your pallas kernel here")


# speedup vs baseline: 29.7174x; 29.7174x over previous
import functools

import jax
import jax.numpy as jnp
from jax import lax
from jax.experimental import pallas as pl
from jax.experimental.pallas import tpu as pltpu
from jax.experimental.pallas import tpu_sc as plsc

_B = 16
_N = 10000
_NPAD = 10240
_D = 128
_E = 320000
_SENTINEL = 1 << 30

_NC = 2
_NS = 16
_NW = _NC * _NS
_CT = 80
_NT = _N // _CT
_EC = _E // _NW


def _prep_body(xp_ref, batch_ref, p_ref, w_ref, src_ref, nb_ref, kvec_ref):
    x = xp_ref[:]
    bat = batch_ref[:]
    p = p_ref[:]

    score = jnp.sum(x * p, axis=1, keepdims=True)
    g_row = lax.broadcasted_iota(jnp.int32, (1, _B), 1).astype(jnp.float32)
    batf = bat.astype(jnp.float32)
    realb = bat < _B
    mask = batf == g_row
    maskf = mask.astype(jnp.float32)

    neg = jnp.float32(-3.0e38)
    smax = jnp.max(jnp.where(mask, score, neg), axis=0, keepdims=True)
    smax = jnp.where(smax > neg * 0.5, smax, 0.0)
    e = jnp.exp(score - jnp.sum(maskf * smax, axis=1, keepdims=True))
    ssum = jnp.sum(maskf * e, axis=0, keepdims=True)
    denom = jnp.sum(maskf * ssum, axis=1, keepdims=True)
    w_ref[:] = jnp.where(realb, e / denom, 0.0)

    jcol = lax.broadcasted_iota(jnp.int32, (_NPAD, 1), 0).astype(jnp.float32)
    lessg = (batf < g_row).astype(jnp.float32)
    cum = jnp.sum(lessg, axis=0, keepdims=True)
    cum_nd = jnp.sum(maskf * cum, axis=1, keepdims=True)
    li = jcol - cum_nd
    even = (li - 2.0 * jnp.floor(li * 0.5)) == 0.0
    evenf = (even & realb).astype(jnp.float32)
    kcum = jnp.sum(lessg * evenf, axis=0, keepdims=True)
    kk = jnp.sum(maskf * evenf, axis=0, keepdims=True)
    K_tot = jnp.sum(evenf, axis=0, keepdims=True)
    dcum = cum - kcum

    gk = jnp.sum((jcol >= kcum).astype(jnp.float32), axis=1, keepdims=True) - 1.0
    ohkf = (gk == g_row).astype(jnp.float32)
    cum_k = jnp.sum(ohkf * cum, axis=1, keepdims=True)
    kcum_k = jnp.sum(ohkf * kcum, axis=1, keepdims=True)
    src_keep = cum_k + (jcol - kcum_k)

    jj = jcol - K_tot
    gd = jnp.sum((jj >= dcum).astype(jnp.float32), axis=1, keepdims=True) - 1.0
    ohdf = (gd == g_row).astype(jnp.float32)
    cum_d = jnp.sum(ohdf * cum, axis=1, keepdims=True)
    kk_d = jnp.sum(ohdf * kk, axis=1, keepdims=True)
    dcum_d = jnp.sum(ohdf * dcum, axis=1, keepdims=True)
    src_drop = cum_d + kk_d + (jj - dcum_d)

    keep = jcol < K_tot
    src_ref[:] = jnp.where(keep, src_keep, src_drop).astype(jnp.int32)
    nb_ref[:] = jnp.where(keep, gk, jnp.float32(_B)).astype(jnp.int32)

    maxb = jnp.max(jnp.where(realb, batf, -1.0), axis=0, keepdims=True)
    kval = jnp.where(maxb == jnp.float32(_B - 1), K_tot,
                     jnp.float32(_SENTINEL)).astype(jnp.int32)
    kvec_ref[:] = jnp.broadcast_to(kval, (1, _B))


def _scale_body(x_ref, w_ref, o_ref):
    o_ref[:] = x_ref[:] * w_ref[:]


_prep = pl.pallas_call(
    _prep_body,
    out_shape=(
        jax.ShapeDtypeStruct((_NPAD, 1), jnp.float32),
        jax.ShapeDtypeStruct((_NPAD, 1), jnp.int32),
        jax.ShapeDtypeStruct((_NPAD, 1), jnp.int32),
        jax.ShapeDtypeStruct((1, _B), jnp.int32),
    ),
)

_scale = pl.pallas_call(
    _scale_body,
    grid=(_NPAD // 128,),
    in_specs=[
        pl.BlockSpec((128, _D), lambda i: (i, 0)),
        pl.BlockSpec((128, 1), lambda i: (i, 0)),
    ],
    out_specs=pl.BlockSpec((128, _D), lambda i: (i, 0)),
    out_shape=jax.ShapeDtypeStruct((_NPAD, _D), jnp.float32),
)


@functools.cache
def _get_sc_kernel():
    @functools.partial(
        pl.kernel,
        mesh=plsc.VectorSubcoreMesh(core_axis_name="c", subcore_axis_name="s"),
        out_type=(
            jax.ShapeDtypeStruct((_N, _D), jnp.float32),
            jax.ShapeDtypeStruct((_E,), jnp.int32),
        ),
        scratch_types=[
            pltpu.VMEM((_CT,), jnp.int32),
            pltpu.VMEM((_CT, _D), jnp.float32),
            pltpu.VMEM((_EC,), jnp.int32),
            pltpu.VMEM((_EC,), jnp.int32),
            pltpu.VMEM((_B,), jnp.int32),
            pltpu.SemaphoreType.DMA,
        ],
    )
    def _sc_gather_edges(xsc_hbm, src_hbm, s_hbm, r_hbm, kvec_hbm,
                         xs_out, s2_out, idx_v, rows_v, sv, rv, kv, sem):
        wid = lax.axis_index("s") * _NC + lax.axis_index("c")

        ebase = wid * _EC
        pltpu.sync_copy(s_hbm.at[pl.ds(ebase, _EC)], sv)
        pltpu.sync_copy(r_hbm.at[pl.ds(ebase, _EC)], rv)
        pltpu.sync_copy(kvec_hbm, kv)
        kvv = kv[...]

        def ebody(i, carry):
            sl = pl.ds(i * 16, 16)
            s = sv[sl]
            r = rv[sl]
            m = (s >= kvv) | (r >= kvv)
            sv[sl] = jnp.where(m, r, s)
            return carry

        lax.fori_loop(0, _EC // 16, ebody, 0)
        pltpu.sync_copy(sv, s2_out.at[pl.ds(ebase, _EC)])

        def nbody(i, carry):
            base = (wid + i * _NW) * _CT
            pltpu.sync_copy(src_hbm.at[pl.ds(base, _CT)], idx_v)
            pltpu.async_copy(xsc_hbm.at[idx_v], rows_v, sem).wait()
            pltpu.sync_copy(rows_v, xs_out.at[pl.ds(base, _CT)])
            return carry

        ntiles = (_NT - wid + _NW - 1) // _NW
        lax.fori_loop(0, ntiles, nbody, 0)

    return _sc_gather_edges


def kernel(x, senders, receivers, batch, p):
    xp = jnp.pad(x, ((0, _NPAD - _N), (0, 0)))
    batch_col = jnp.pad(batch, (0, _NPAD - _N),
                        constant_values=_B).reshape(_NPAD, 1)
    p_row = p.reshape(1, _D)

    w_col, src_col, nb_col, kvec = _prep(xp, batch_col, p_row)
    xsc = _scale(xp, w_col)

    src_flat = src_col.reshape(_NPAD)
    xs, senders2 = _get_sc_kernel()(xsc, src_flat, senders, receivers,
                                    kvec.reshape(_B))
    new_batch = nb_col.reshape(_NPAD)[:_N]
    return (xs, senders2, receivers, new_batch)


# speedup vs baseline: 49.2426x; 1.6570x over previous
import functools

import jax
import jax.numpy as jnp
from jax import lax
from jax.experimental import pallas as pl
from jax.experimental.pallas import tpu as pltpu
from jax.experimental.pallas import tpu_sc as plsc

_B = 16
_N = 10000
_D = 128
_E = 320000
_SENTINEL = 1 << 30

_NC = 2
_NS = 16
_NW = _NC * _NS
_CT = 80
_NT = _N // _CT
_EC = _E // _NW


def _prep_body(x_ref, batch_ref, p_ref, src_ref, nb_ref, kvec_ref,
               xsc_ref):
    x = x_ref[:]
    bat = batch_ref[:]
    p = p_ref[:]

    score = jnp.sum(x * p, axis=1, keepdims=True)
    g_row = lax.broadcasted_iota(jnp.int32, (1, _B), 1).astype(jnp.float32)
    batf = bat.astype(jnp.float32)
    mask = batf == g_row
    maskf = mask.astype(jnp.float32)

    neg = jnp.float32(-3.0e38)
    smax = jnp.max(jnp.where(mask, score, neg), axis=0, keepdims=True)
    smax = jnp.where(smax > neg * 0.5, smax, 0.0)
    e = jnp.exp(score - jnp.sum(maskf * smax, axis=1, keepdims=True))
    ssum = jnp.sum(maskf * e, axis=0, keepdims=True)
    denom = jnp.sum(maskf * ssum, axis=1, keepdims=True)
    xsc_ref[:] = x * (e / denom)

    jcol = lax.broadcasted_iota(jnp.int32, (_N, 1), 0).astype(jnp.float32)
    lessg = (batf < g_row).astype(jnp.float32)
    cum = jnp.sum(lessg, axis=0, keepdims=True)
    cum_nd = jnp.sum(maskf * cum, axis=1, keepdims=True)
    li = jcol - cum_nd
    even = (li - 2.0 * jnp.floor(li * 0.5)) == 0.0
    evenf = even.astype(jnp.float32)
    kcum = jnp.sum(lessg * evenf, axis=0, keepdims=True)
    kk = jnp.sum(maskf * evenf, axis=0, keepdims=True)
    K_tot = jnp.sum(evenf, axis=0, keepdims=True)
    dcum = cum - kcum

    gk = jnp.sum((jcol >= kcum).astype(jnp.float32), axis=1, keepdims=True) - 1.0
    ohkf = (gk == g_row).astype(jnp.float32)
    cum_k = jnp.sum(ohkf * cum, axis=1, keepdims=True)
    kcum_k = jnp.sum(ohkf * kcum, axis=1, keepdims=True)
    src_keep = cum_k + (jcol - kcum_k)

    jj = jcol - K_tot
    gd = jnp.sum((jj >= dcum).astype(jnp.float32), axis=1, keepdims=True) - 1.0
    ohdf = (gd == g_row).astype(jnp.float32)
    cum_d = jnp.sum(ohdf * cum, axis=1, keepdims=True)
    kk_d = jnp.sum(ohdf * kk, axis=1, keepdims=True)
    dcum_d = jnp.sum(ohdf * dcum, axis=1, keepdims=True)
    src_drop = cum_d + kk_d + (jj - dcum_d)

    keep = jcol < K_tot
    src_ref[:] = jnp.where(keep, src_keep, src_drop).astype(jnp.int32)
    nb_ref[:] = jnp.where(keep, gk, jnp.float32(_B)).astype(jnp.int32)

    maxb = jnp.max(batf, axis=0, keepdims=True)
    kval = jnp.where(maxb == jnp.float32(_B - 1), K_tot,
                     jnp.float32(_SENTINEL)).astype(jnp.int32)
    kvec_ref[:] = jnp.broadcast_to(kval, (1, _B))


_prep = pl.pallas_call(
    _prep_body,
    out_shape=(
        jax.ShapeDtypeStruct((_N, 1), jnp.int32),
        jax.ShapeDtypeStruct((_N, 1), jnp.int32),
        jax.ShapeDtypeStruct((1, _B), jnp.int32),
        jax.ShapeDtypeStruct((_N, _D), jnp.float32),
    ),
)


@functools.cache
def _get_sc_kernel():
    @functools.partial(
        pl.kernel,
        mesh=plsc.VectorSubcoreMesh(core_axis_name="c", subcore_axis_name="s"),
        out_type=(
            jax.ShapeDtypeStruct((_N, _D), jnp.float32),
            jax.ShapeDtypeStruct((_E,), jnp.int32),
        ),
        scratch_types=[
            pltpu.VMEM((_CT,), jnp.int32),
            pltpu.VMEM((_CT, _D), jnp.float32),
            pltpu.VMEM((_EC,), jnp.int32),
            pltpu.VMEM((_EC,), jnp.int32),
            pltpu.VMEM((_B,), jnp.int32),
            pltpu.SemaphoreType.DMA,
        ],
    )
    def _sc_gather_edges(xsc_hbm, src_hbm, s_hbm, r_hbm, kvec_hbm,
                         xs_out, s2_out, idx_v, rows_v, sv, rv, kv, sem):
        wid = lax.axis_index("s") * _NC + lax.axis_index("c")

        ebase = wid * _EC
        pltpu.sync_copy(s_hbm.at[pl.ds(ebase, _EC)], sv)
        pltpu.sync_copy(r_hbm.at[pl.ds(ebase, _EC)], rv)
        pltpu.sync_copy(kvec_hbm, kv)
        kvv = kv[...]

        def ebody(i, carry):
            sl = pl.ds(i * 16, 16)
            s = sv[sl]
            r = rv[sl]
            m = (s >= kvv) | (r >= kvv)
            sv[sl] = jnp.where(m, r, s)
            return carry

        lax.fori_loop(0, _EC // 16, ebody, 0)
        pltpu.sync_copy(sv, s2_out.at[pl.ds(ebase, _EC)])

        def nbody(i, carry):
            base = (wid + i * _NW) * _CT
            pltpu.sync_copy(src_hbm.at[pl.ds(base, _CT)], idx_v)
            pltpu.async_copy(xsc_hbm.at[idx_v], rows_v, sem).wait()
            pltpu.sync_copy(rows_v, xs_out.at[pl.ds(base, _CT)])
            return carry

        ntiles = (_NT - wid + _NW - 1) // _NW
        lax.fori_loop(0, ntiles, nbody, 0)

    return _sc_gather_edges


def kernel(x, senders, receivers, batch, p):
    batch_col = batch.reshape(_N, 1)
    p_row = p.reshape(1, _D)

    src_col, nb_col, kvec, xsc = _prep(x, batch_col, p_row)

    src_flat = src_col.reshape(_N)
    xs, senders2 = _get_sc_kernel()(xsc, src_flat, senders, receivers,
                                    kvec.reshape(_B))
    new_batch = nb_col.reshape(_N)
    return (xs, senders2, receivers, new_batch)


# speedup vs baseline: 54.0827x; 1.0983x over previous
import functools

import jax
import jax.numpy as jnp
from jax import lax
from jax.experimental import pallas as pl
from jax.experimental.pallas import tpu as pltpu
from jax.experimental.pallas import tpu_sc as plsc

_B = 16
_N = 10000
_D = 128
_E = 320000
_SENTINEL = 1 << 30

_NC = 2
_NS = 16
_NW = _NC * _NS
_CW = 312
_TB = _NW * _CW
_TAIL = _N - _TB
_EC = _E // _NW


def _prep_body(x_ref, batch_ref, p_ref, src_ref, nb_ref, kvec_ref,
               xsc_ref):
    x = x_ref[:]
    bat = batch_ref[:]
    p = p_ref[:]

    score = jnp.sum(x * p, axis=1, keepdims=True)
    g_row = lax.broadcasted_iota(jnp.int32, (1, _B), 1).astype(jnp.float32)
    batf = bat.astype(jnp.float32)
    mask = batf == g_row
    maskf = mask.astype(jnp.float32)

    neg = jnp.float32(-3.0e38)
    smax = jnp.max(jnp.where(mask, score, neg), axis=0, keepdims=True)
    smax = jnp.where(smax > neg * 0.5, smax, 0.0)
    e = jnp.exp(score - jnp.sum(maskf * smax, axis=1, keepdims=True))
    ssum = jnp.sum(maskf * e, axis=0, keepdims=True)
    denom = jnp.sum(maskf * ssum, axis=1, keepdims=True)
    xsc_ref[:] = x * (e / denom)

    jcol = lax.broadcasted_iota(jnp.int32, (_N, 1), 0).astype(jnp.float32)
    lessg = (batf < g_row).astype(jnp.float32)
    cum = jnp.sum(lessg, axis=0, keepdims=True)
    cum_nd = jnp.sum(maskf * cum, axis=1, keepdims=True)
    li = jcol - cum_nd
    even = (li - 2.0 * jnp.floor(li * 0.5)) == 0.0
    evenf = even.astype(jnp.float32)
    kcum = jnp.sum(lessg * evenf, axis=0, keepdims=True)
    kk = jnp.sum(maskf * evenf, axis=0, keepdims=True)
    K_tot = jnp.sum(evenf, axis=0, keepdims=True)
    dcum = cum - kcum

    gk = jnp.sum((jcol >= kcum).astype(jnp.float32), axis=1, keepdims=True) - 1.0
    ohkf = (gk == g_row).astype(jnp.float32)
    cum_k = jnp.sum(ohkf * cum, axis=1, keepdims=True)
    kcum_k = jnp.sum(ohkf * kcum, axis=1, keepdims=True)
    src_keep = cum_k + (jcol - kcum_k)

    jj = jcol - K_tot
    gd = jnp.sum((jj >= dcum).astype(jnp.float32), axis=1, keepdims=True) - 1.0
    ohdf = (gd == g_row).astype(jnp.float32)
    cum_d = jnp.sum(ohdf * cum, axis=1, keepdims=True)
    kk_d = jnp.sum(ohdf * kk, axis=1, keepdims=True)
    dcum_d = jnp.sum(ohdf * dcum, axis=1, keepdims=True)
    src_drop = cum_d + kk_d + (jj - dcum_d)

    keep = jcol < K_tot
    src_ref[:] = jnp.where(keep, src_keep, src_drop).astype(jnp.int32)
    nb_ref[:] = jnp.where(keep, gk, jnp.float32(_B)).astype(jnp.int32)

    maxb = jnp.max(batf, axis=0, keepdims=True)
    kval = jnp.where(maxb == jnp.float32(_B - 1), K_tot,
                     jnp.float32(_SENTINEL)).astype(jnp.int32)
    kvec_ref[:] = jnp.broadcast_to(kval, (1, _B))


_prep = pl.pallas_call(
    _prep_body,
    out_shape=(
        jax.ShapeDtypeStruct((_N, 1), jnp.int32),
        jax.ShapeDtypeStruct((_N, 1), jnp.int32),
        jax.ShapeDtypeStruct((1, _B), jnp.int32),
        jax.ShapeDtypeStruct((_N, _D), jnp.float32),
    ),
)


@functools.cache
def _get_sc_kernel():
    @functools.partial(
        pl.kernel,
        mesh=plsc.VectorSubcoreMesh(core_axis_name="c", subcore_axis_name="s"),
        out_type=(
            jax.ShapeDtypeStruct((_N, _D), jnp.float32),
            jax.ShapeDtypeStruct((_E,), jnp.int32),
        ),
        scratch_types=[
            pltpu.VMEM((_CW,), jnp.int32),
            pltpu.VMEM((_CW, _D), jnp.float32),
            pltpu.VMEM((_EC,), jnp.int32),
            pltpu.VMEM((_EC,), jnp.int32),
            pltpu.VMEM((_B,), jnp.int32),
            pltpu.SemaphoreType.DMA,
            pltpu.SemaphoreType.DMA,
            pltpu.SemaphoreType.DMA,
        ],
    )
    def _sc_gather_edges(xsc_hbm, src_hbm, s_hbm, r_hbm, kvec_hbm,
                         xs_out, s2_out, idx_v, rows_v, sv, rv, kv,
                         esem, gsem, osem):
        wid = lax.axis_index("s") * _NC + lax.axis_index("c")
        ebase = wid * _EC
        nbase = wid * _CW

        e1 = pltpu.async_copy(s_hbm.at[pl.ds(ebase, _EC)], sv, esem)
        e2 = pltpu.async_copy(r_hbm.at[pl.ds(ebase, _EC)], rv, esem)
        pltpu.sync_copy(kvec_hbm, kv)

        pltpu.sync_copy(src_hbm.at[pl.ds(nbase, _CW)], idx_v)
        gathers = []
        for off in (0, 104, 208):
            gathers.append(pltpu.async_copy(
                xsc_hbm.at[idx_v.at[pl.ds(off, 104)]],
                rows_v.at[pl.ds(off, 104)], gsem))

        e1.wait()
        e2.wait()
        kvv = kv[...]

        def ebody(i, carry):
            sl = pl.ds(i * 16, 16)
            s = sv[sl]
            r = rv[sl]
            m = (s >= kvv) | (r >= kvv)
            sv[sl] = jnp.where(m, r, s)
            return carry

        lax.fori_loop(0, _EC // 16, ebody, 0)
        eo = pltpu.async_copy(sv, s2_out.at[pl.ds(ebase, _EC)], osem)

        for g in gathers:
            g.wait()
        pltpu.sync_copy(rows_v, xs_out.at[pl.ds(nbase, _CW)])

        @pl.when(wid == 0)
        def _():
            pltpu.sync_copy(src_hbm.at[pl.ds(_TB, _TAIL)],
                            idx_v.at[pl.ds(0, _TAIL)])
            pltpu.async_copy(xsc_hbm.at[idx_v.at[pl.ds(0, _TAIL)]],
                             rows_v.at[pl.ds(0, _TAIL)], gsem).wait()
            pltpu.sync_copy(rows_v.at[pl.ds(0, _TAIL)],
                            xs_out.at[pl.ds(_TB, _TAIL)])

        eo.wait()

    return _sc_gather_edges


def kernel(x, senders, receivers, batch, p):
    batch_col = batch.reshape(_N, 1)
    p_row = p.reshape(1, _D)

    src_col, nb_col, kvec, xsc = _prep(x, batch_col, p_row)

    src_flat = src_col.reshape(_N)
    xs, senders2 = _get_sc_kernel()(xsc, src_flat, senders, receivers,
                                    kvec.reshape(_B))
    new_batch = nb_col.reshape(_N)
    return (xs, senders2, receivers, new_batch)
